# Initial kernel scaffold; baseline (speedup 1.0000x reference)
import jax
import jax.numpy as jnp
from jax.experimental import pallas as pl


def kernel(x, write_mask, W1_0, b1_0, W2_0, b2_0, g_0, be_0, W1_1, b1_1, W2_1, b2_1, g_1, be_1, Wf, bf, g_ln, b_ln, K_fast, V_fast, K_deep, V_deep):
    raise NotImplementedError("write your pallas kernel here")


# speedup vs baseline: 2.2426x; 2.2426x over previous
import functools
import math

import jax
import jax.numpy as jnp
from jax.experimental import pallas as pl
from jax.experimental.pallas import tpu as pltpu

D = 512
M = 8192
TOPK = 8
LR_FAST = 1.0
LR_DEEP = 0.1
DECAY = 0.9995
MT = 512
MTU = 1024
CW = 512

_HIGH = jax.lax.Precision.HIGHEST
NEG = -3.0e38


def _layernorm(x, g, b):
    mu = jnp.mean(x, axis=-1, keepdims=True)
    var = jnp.mean((x - mu) ** 2, axis=-1, keepdims=True)
    return (x - mu) * jax.lax.rsqrt(var + 1e-5) * g + b


def _mamba_body(x_ref, w10, b10, w20, b20, g0, be0, w11, b11, w21, b21, g1,
                be1, o_ref):
    h = x_ref[...]
    for (w1, b1, w2, b2, g, be) in ((w10, b10, w20, b20, g0, be0),
                                    (w11, b11, w21, b21, g1, be1)):
        a = jnp.dot(h, w1[...],
                    preferred_element_type=jnp.float32) + b1[...]
        a = jax.nn.gelu(a)
        hh = jnp.dot(a, w2[...],
                     preferred_element_type=jnp.float32) + b2[...]
        hh = _layernorm(hh, g[...], be[...])
        h = h + hh
    o_ref[...] = h


def _mamba(xf, W1_0, b1_0, W2_0, b2_0, g_0, be_0, W1_1, b1_1, W2_1, b2_1,
           g_1, be_1):
    n = xf.shape[0]
    blk = 512 if n % 512 == 0 else n
    grid = (n // blk,)
    full = lambda shp: pl.BlockSpec(shp, lambda i: tuple(0 for _ in shp))
    return pl.pallas_call(
        _mamba_body,
        grid=grid,
        in_specs=[pl.BlockSpec((blk, D), lambda i: (i, 0))] + [
            full(w.shape) for w in (W1_0, b1_0, W2_0, b2_0, g_0, be_0,
                                    W1_1, b1_1, W2_1, b2_1, g_1, be_1)],
        out_specs=pl.BlockSpec((blk, D), lambda i: (i, 0)),
        out_shape=jax.ShapeDtypeStruct((n, D), jnp.float32),
        compiler_params=pltpu.CompilerParams(
            dimension_semantics=("arbitrary",)),
    )(xf, W1_0, b1_0, W2_0, b2_0, g_0, be_0, W1_1, b1_1, W2_1, b2_1, g_1,
      be_1)


def _select_body(h_ref, ak_ref, av_ref, s_ref, v_ref, idx_ref, gk_ref,
                 gv_ref, sim_ref, m1_ref, v8_ref, den_ref, *, nb, nmt):
    p = pl.program_id(0)
    j = pl.program_id(1)
    scale = s_ref[0]
    vscale = s_ref[1]

    @pl.when(p == 0)
    def _phase_sim():
        sim_ref[:, pl.ds(j * MT, MT)] = scale * jax.lax.dot_general(
            h_ref[...], ak_ref[...], (((1,), (1,)), ((), ())),
            preferred_element_type=jnp.float32)

    @pl.when((p == 1) & (j == 0))
    def _phase_topk():
        nch = (nmt * MT) // CW

        def masked_max(thr):
            def body(c, cur):
                ch = sim_ref[:, pl.ds(c * CW, CW)]
                ch = jnp.where(ch < thr, ch, NEG)
                return jnp.maximum(cur, jnp.max(ch, axis=1, keepdims=True))
            return jax.lax.fori_loop(0, nch, body,
                                     jnp.full((nb, 1), NEG, jnp.float32))

        m1 = masked_max(jnp.full((nb, 1), 3.0e38, jnp.float32))
        thr = jax.lax.fori_loop(0, TOPK - 1, lambda k, t: masked_max(t), m1)
        m1_ref[...] = m1
        v8_ref[...] = thr

        def argbody(c, cur):
            ch = sim_ref[:, pl.ds(c * CW, CW)]
            io = jax.lax.broadcasted_iota(jnp.int32, (nb, CW), 1) + c * CW
            cand = jnp.min(jnp.where(ch == m1, io, jnp.int32(2 ** 30)),
                           axis=1, keepdims=True)
            return jnp.minimum(cur, cand)
        idx_ref[...] = jax.lax.fori_loop(
            0, nch, argbody, jnp.full((nb, 1), 2 ** 30, jnp.int32))

    @pl.when(p == 1)
    def _phase_read():
        simt = sim_ref[:, pl.ds(j * MT, MT)]
        pmat = jnp.where(simt >= v8_ref[...],
                         jnp.exp(simt - m1_ref[...]), 0.0)
        io = jax.lax.broadcasted_iota(jnp.int32, (nb, MT), 1) + j * MT
        smat = (idx_ref[...] == io).astype(jnp.float32)
        pv = jax.lax.dot_general(pmat, av_ref[...], (((1,), (0,)), ((), ())),
                                 precision=_HIGH,
                                 preferred_element_type=jnp.float32)
        gk = jax.lax.dot_general(smat, ak_ref[...], (((1,), (0,)), ((), ())),
                                 precision=_HIGH,
                                 preferred_element_type=jnp.float32)
        gv = jax.lax.dot_general(smat, av_ref[...], (((1,), (0,)), ((), ())),
                                 precision=_HIGH,
                                 preferred_element_type=jnp.float32)
        dloc = jnp.sum(pmat, axis=1, keepdims=True)

        @pl.when(j == 0)
        def _init():
            v_ref[...] = pv
            gk_ref[...] = gk
            gv_ref[...] = gv
            den_ref[...] = dloc

        @pl.when(j > 0)
        def _acc():
            v_ref[...] += pv
            gk_ref[...] += gk
            gv_ref[...] += gv
            den_ref[...] += dloc

        @pl.when(j == nmt - 1)
        def _fin():
            v_ref[...] = v_ref[...] * (vscale / den_ref[...])


def _select(h_t, a_k, a_v, scal):
    nb = h_t.shape[0]
    m = a_k.shape[0]
    nmt = m // MT
    body = functools.partial(_select_body, nb=nb, nmt=nmt)
    return pl.pallas_call(
        body,
        grid=(2, nmt),
        in_specs=[
            pl.BlockSpec((nb, D), lambda p, j: (0, 0)),
            pl.BlockSpec((MT, D), lambda p, j: (j, 0)),
            pl.BlockSpec((MT, D), lambda p, j: (jnp.where(p == 1, j, 0), 0)),
            pl.BlockSpec(memory_space=pltpu.SMEM),
        ],
        out_specs=[
            pl.BlockSpec((nb, D), lambda p, j: (0, 0)),
            pl.BlockSpec((nb, 1), lambda p, j: (0, 0)),
            pl.BlockSpec((nb, D), lambda p, j: (0, 0)),
            pl.BlockSpec((nb, D), lambda p, j: (0, 0)),
        ],
        out_shape=[
            jax.ShapeDtypeStruct((nb, D), jnp.float32),
            jax.ShapeDtypeStruct((nb, 1), jnp.int32),
            jax.ShapeDtypeStruct((nb, D), jnp.float32),
            jax.ShapeDtypeStruct((nb, D), jnp.float32),
        ],
        scratch_shapes=[
            pltpu.VMEM((nb, m), jnp.float32),
            pltpu.VMEM((nb, 1), jnp.float32),
            pltpu.VMEM((nb, 1), jnp.float32),
            pltpu.VMEM((nb, 1), jnp.float32),
        ],
        compiler_params=pltpu.CompilerParams(
            dimension_semantics=("arbitrary", "arbitrary"),
            vmem_limit_bytes=120 * 1024 * 1024),
    )(h_t, a_k, a_v, scal)


def _fused_body(h_ref, vf_ref, vd_ref, wf_ref, bf_ref, g_ref, b_ref,
                gkf_ref, gvf_ref, gkd_ref, gvd_ref, s_ref,
                out_ref, ukf_ref, uvf_ref, ukd_ref, uvd_ref):
    h = h_ref[...]
    v = 0.5 * (vf_ref[...] + vd_ref[...])
    fused = (jnp.dot(h, wf_ref[:D, :],
                     preferred_element_type=jnp.float32)
             + jnp.dot(v, wf_ref[D:, :],
                       preferred_element_type=jnp.float32) + bf_ref[...])
    fused = _layernorm(fused + h, g_ref[...], b_ref[...])
    out_ref[...] = fused
    cf, lf, cd, ld = s_ref[0], s_ref[1], s_ref[2], s_ref[3]
    ukf_ref[...] = cf * h - lf * gkf_ref[...]
    uvf_ref[...] = cf * fused - lf * gvf_ref[...]
    ukd_ref[...] = cd * h - ld * gkd_ref[...]
    uvd_ref[...] = cd * fused - ld * gvd_ref[...]


def _fused(h_t, v_f, v_d, Wf, bf, g_ln, b_ln, gkf, gvf, gkd, gvd, scal):
    nb = h_t.shape[0]
    full = lambda shp: pl.BlockSpec(shp, lambda: tuple(0 for _ in shp))
    outs = [jax.ShapeDtypeStruct((nb, D), jnp.float32)] * 5
    return pl.pallas_call(
        _fused_body,
        in_specs=[full((nb, D)), full((nb, D)), full((nb, D)),
                  full((2 * D, D)), full((D,)), full((D,)), full((D,)),
                  full((nb, D)), full((nb, D)), full((nb, D)), full((nb, D)),
                  pl.BlockSpec(memory_space=pltpu.SMEM)],
        out_specs=[full((nb, D))] * 5,
        out_shape=outs,
    )(h_t, v_f, v_d, Wf, bf, g_ln, b_ln, gkf, gvf, gkd, gvd, scal)


def _update_body(idx_ref, uk_ref, uv_ref, ak_ref, av_ref, ako_ref, avo_ref,
                 *, nb):
    j = pl.program_id(0)
    io = jax.lax.broadcasted_iota(jnp.int32, (nb, MTU), 1) + j * MTU
    smat = (idx_ref[...] == io).astype(jnp.float32)
    ako_ref[...] = ak_ref[...] + jax.lax.dot_general(
        smat, uk_ref[...], (((0,), (0,)), ((), ())), precision=_HIGH,
        preferred_element_type=jnp.float32)
    avo_ref[...] = av_ref[...] + jax.lax.dot_general(
        smat, uv_ref[...], (((0,), (0,)), ((), ())), precision=_HIGH,
        preferred_element_type=jnp.float32)


def _update(idx, u_k, u_v, a_k, a_v):
    nb = u_k.shape[0]
    m = a_k.shape[0]
    body = functools.partial(_update_body, nb=nb)
    return pl.pallas_call(
        body,
        grid=(m // MTU,),
        in_specs=[
            pl.BlockSpec((nb, 1), lambda j: (0, 0)),
            pl.BlockSpec((nb, D), lambda j: (0, 0)),
            pl.BlockSpec((nb, D), lambda j: (0, 0)),
            pl.BlockSpec((MTU, D), lambda j: (j, 0)),
            pl.BlockSpec((MTU, D), lambda j: (j, 0)),
        ],
        out_specs=[
            pl.BlockSpec((MTU, D), lambda j: (j, 0)),
            pl.BlockSpec((MTU, D), lambda j: (j, 0)),
        ],
        out_shape=[jax.ShapeDtypeStruct((m, D), jnp.float32)] * 2,
        input_output_aliases={3: 0, 4: 1},
        compiler_params=pltpu.CompilerParams(
            dimension_semantics=("arbitrary",)),
    )(idx, u_k, u_v, a_k, a_v)


def kernel(x, write_mask, W1_0, b1_0, W2_0, b2_0, g_0, be_0, W1_1, b1_1,
           W2_1, b2_1, g_1, be_1, Wf, bf, g_ln, b_ln, K_fast, V_fast,
           K_deep, V_deep):
    B, T, d = x.shape
    xf = x.reshape(B * T, d)
    h = _mamba(xf, W1_0, b1_0, W2_0, b2_0, g_0, be_0, W1_1, b1_1, W2_1,
               b2_1, g_1, be_1).reshape(B, T, d)

    anyb = jnp.any(write_mask, axis=0).astype(jnp.float32)
    invsq = jnp.float32(1.0 / math.sqrt(d))

    akf, avf, akd, avd = K_fast, V_fast, K_deep, V_deep
    p_t = jnp.float32(0.0)
    outs = []
    for t in range(T):
        h_t = h[:, t, :]
        decp = DECAY ** p_t
        scal = jnp.stack([decp * invsq, decp])
        vf, idxf, gkf, gvf = _select(h_t, akf, avf, scal)
        vd, idxd, gkd, gvd = _select(h_t, akd, avd, scal)
        a_t = anyb[t]
        cscale = a_t / (decp * DECAY)
        uscal = jnp.stack([LR_FAST * cscale, LR_FAST * a_t,
                           LR_DEEP * cscale, LR_DEEP * a_t])
        out_t, ukf, uvf, ukd, uvd = _fused(h_t, vf, vd, Wf, bf, g_ln, b_ln,
                                           gkf, gvf, gkd, gvd, uscal)
        outs.append(out_t)
        if t + 1 < T:
            akf, avf = _update(idxf, ukf, uvf, akf, avf)
            akd, avd = _update(idxd, ukd, uvd, akd, avd)
            p_t = p_t + a_t
    return jnp.stack(outs, axis=1)
